# baseline (speedup 1.0000x reference)
import functools

import jax
import jax.numpy as jnp
from jax import lax
from jax.experimental import pallas as pl
from jax.experimental.pallas import tpu as pltpu
from jax.experimental.pallas import tpu_sc as plsc

_VOCAB = 30522
_B = 16384
_S = 200
_NW = 32
_BPW = _B // _NW
_G = _BPW // 16
_U = 8
_TBLK = 4096
_TLEN = 32768

_HIGHEST = jax.lax.Precision.HIGHEST


def _table_body(et_ref, w1_ref, b1col_ref, wc_ref, bc2_ref, out_ref,
                wf_s, bf_s):
    @pl.when(pl.program_id(0) == 0)
    def _():
        wf_s[...] = jax.lax.dot_general(
            w1_ref[...], wc_ref[...], (((1,), (0,)), ((), ())),
            preferred_element_type=jnp.float32, precision=_HIGHEST)
        bf_s[...] = jax.lax.dot_general(
            b1col_ref[...], wc_ref[...], (((0,), (0,)), ((), ())),
            preferred_element_type=jnp.float32,
            precision=_HIGHEST) + bc2_ref[...]

    tt = jax.lax.dot_general(
        wf_s[...], et_ref[...], (((0,), (0,)), ((), ())),
        preferred_element_type=jnp.float32, precision=_HIGHEST)
    t0 = (tt[0:1, :] + bf_s[0:1, 0:1]) * (1.0 / _S)
    t1 = (tt[1:2, :] + bf_s[0:1, 1:2]) * (1.0 / _S)
    u0 = jax.lax.bitcast_convert_type(
        t0.astype(jnp.bfloat16), jnp.uint16).astype(jnp.int32)
    u1 = jax.lax.bitcast_convert_type(
        t1.astype(jnp.bfloat16), jnp.uint16).astype(jnp.int32)
    out_ref[...] = jnp.reshape(u0 | (u1 << 16), (_TBLK,))


_TBLK = 4096

_table_kernel = pl.pallas_call(
    _table_body,
    grid=(_TLEN // _TBLK,),
    in_specs=[
        pl.BlockSpec((32, _TBLK), lambda i: (0, i)),
        pl.BlockSpec((32, 128), lambda i: (0, 0)),
        pl.BlockSpec((128, 1), lambda i: (0, 0)),
        pl.BlockSpec((128, 2), lambda i: (0, 0)),
        pl.BlockSpec((1, 2), lambda i: (0, 0)),
    ],
    out_specs=pl.BlockSpec((_TBLK,), lambda i: (i,)),
    out_shape=jax.ShapeDtypeStruct((_TLEN,), jnp.int32),
    scratch_shapes=[
        pltpu.VMEM((32, 2), jnp.float32),
        pltpu.VMEM((1, 2), jnp.float32),
    ],
)


def _sc_bag_body(tbl_hbm, texts_hbm, out_hbm, tbl_v, idx0, idx1,
                 out0_v, out1_v, sem_t, sem0, sem1):
    cid = lax.axis_index("c")
    sid = lax.axis_index("s")
    wid = sid * 2 + cid
    base = wid * _BPW

    tbl_copy = pltpu.async_copy(tbl_hbm, tbl_v, sem_t)
    bufs = (idx0, idx1)
    sems = (sem0, sem1)
    copies = [None, None]
    gbase = wid * _G
    copies[0] = pltpu.async_copy(
        texts_hbm.at[pl.ds(gbase * 16 * _S, 16 * _S)], idx0, sem0)
    tbl_copy.wait()

    offs = lax.iota(jnp.int32, 16) * _S


    for g in range(_G):
        cur = g & 1
        if g + 1 < _G:
            nxt = (g + 1) & 1
            copies[nxt] = pltpu.async_copy(
                texts_hbm.at[pl.ds((gbase + g + 1) * 16 * _S, 16 * _S)],
                bufs[nxt], sems[nxt])
        copies[cur].wait()
        iref = bufs[cur]

        def body(i, carry, iref=iref):
            a0, a1, tv = carry
            for j in range(_U):
                iv = plsc.load_gather(iref, [tv])
                w = plsc.load_gather(tbl_v, [iv])
                v0 = plsc.bitcast(lax.shift_left(w, 16), jnp.float32)
                v1 = plsc.bitcast(
                    lax.bitwise_and(w, jnp.int32(-65536)), jnp.float32)
                a0 = a0 + v0
                a1 = a1 + v1
                tv = tv + 1
            return (a0, a1, tv)

        zero = jnp.zeros((16,), jnp.float32)
        acc0, acc1, _ = lax.fori_loop(0, _S // _U, body, (zero, zero, offs))
        out0_v[pl.ds(g * 16, 16)] = acc0
        out1_v[pl.ds(g * 16, 16)] = acc1

    pltpu.sync_copy(out0_v, out_hbm.at[pl.ds(base, _BPW)])
    pltpu.sync_copy(out1_v, out_hbm.at[pl.ds(_B + base, _BPW)])


_sc_bag = functools.partial(
    pl.kernel,
    out_type=jax.ShapeDtypeStruct((2 * _B,), jnp.float32),
    mesh=plsc.VectorSubcoreMesh(core_axis_name="c", subcore_axis_name="s"),
    compiler_params=pltpu.CompilerParams(needs_layout_passes=False),
    scratch_types=[
        pltpu.VMEM((_TLEN,), jnp.int32),
        pltpu.VMEM((16 * _S,), jnp.int32),
        pltpu.VMEM((16 * _S,), jnp.int32),
        pltpu.VMEM((_BPW,), jnp.float32),
        pltpu.VMEM((_BPW,), jnp.float32),
        pltpu.SemaphoreType.DMA,
        pltpu.SemaphoreType.DMA,
        pltpu.SemaphoreType.DMA,
    ],
)(_sc_bag_body)


def kernel(texts, embed, W1, b1, Wc, bc):
    tbl = _table_kernel(embed.T, W1, b1.reshape(-1, 1), Wc,
                        bc.reshape(1, -1))
    out = _sc_bag(tbl, texts.reshape(-1))
    return out.reshape(2, _B).T


# speedup vs baseline: 1.1133x; 1.1133x over previous
import functools

import jax
import jax.numpy as jnp
from jax import lax
from jax.experimental import pallas as pl
from jax.experimental.pallas import tpu as pltpu
from jax.experimental.pallas import tpu_sc as plsc

_VOCAB = 30522
_B = 16384
_S = 200
_NW = 32
_BPW = _B // _NW
_G = _BPW // 16
_U = 8
_SLAB = 8
_TBLK = 4096
_TLEN = 32768

_HIGHEST = jax.lax.Precision.HIGHEST


def _table_body(et_ref, w1_ref, b1col_ref, wc_ref, bc2_ref, out_ref,
                wf_s, bf_s):
    @pl.when(pl.program_id(0) == 0)
    def _():
        wf_s[...] = jax.lax.dot_general(
            w1_ref[...], wc_ref[...], (((1,), (0,)), ((), ())),
            preferred_element_type=jnp.float32, precision=_HIGHEST)
        bf_s[...] = jax.lax.dot_general(
            b1col_ref[...], wc_ref[...], (((0,), (0,)), ((), ())),
            preferred_element_type=jnp.float32,
            precision=_HIGHEST) + bc2_ref[...]

    tt = jax.lax.dot_general(
        wf_s[...], et_ref[...], (((0,), (0,)), ((), ())),
        preferred_element_type=jnp.float32, precision=_HIGHEST)
    t0 = (tt[0:1, :] + bf_s[0:1, 0:1]) * (1.0 / _S)
    t1 = (tt[1:2, :] + bf_s[0:1, 1:2]) * (1.0 / _S)
    u0 = jax.lax.bitcast_convert_type(
        t0.astype(jnp.bfloat16), jnp.uint16).astype(jnp.int32)
    u1 = jax.lax.bitcast_convert_type(
        t1.astype(jnp.bfloat16), jnp.uint16).astype(jnp.int32)
    out_ref[...] = jnp.reshape(u0 | (u1 << 16), (_TBLK,))


_TBLK = 4096

_table_kernel = pl.pallas_call(
    _table_body,
    grid=(_TLEN // _TBLK,),
    in_specs=[
        pl.BlockSpec((32, _TBLK), lambda i: (0, i)),
        pl.BlockSpec((32, 128), lambda i: (0, 0)),
        pl.BlockSpec((128, 1), lambda i: (0, 0)),
        pl.BlockSpec((128, 2), lambda i: (0, 0)),
        pl.BlockSpec((1, 2), lambda i: (0, 0)),
    ],
    out_specs=pl.BlockSpec((_TBLK,), lambda i: (i,)),
    out_shape=jax.ShapeDtypeStruct((_TLEN,), jnp.int32),
    scratch_shapes=[
        pltpu.VMEM((32, 2), jnp.float32),
        pltpu.VMEM((1, 2), jnp.float32),
    ],
)


def _sc_bag_body(tbl_hbm, texts_hbm, out_hbm, tbl_v, sl0, sl1,
                 out0_v, out1_v, sem_t, sem0, sem1):
    cid = lax.axis_index("c")
    sid = lax.axis_index("s")
    wid = sid * 2 + cid
    base = wid * _BPW

    tbl_copy = pltpu.async_copy(tbl_hbm, tbl_v, sem_t)
    bufs = (sl0, sl1)
    sems = (sem0, sem1)
    copies = [None, None]
    gbase = wid * _G
    copies[0] = pltpu.async_copy(
        texts_hbm.at[pl.ds(gbase, _SLAB), :], sl0, sem0)
    tbl_copy.wait()

    offs = lax.iota(jnp.int32, 16) * _S
    zero = jnp.zeros((16,), jnp.float32)

    for sb in range(_G // _SLAB):
        cur = sb & 1
        if sb + 1 < _G // _SLAB:
            nxt = (sb + 1) & 1
            copies[nxt] = pltpu.async_copy(
                texts_hbm.at[pl.ds(gbase + (sb + 1) * _SLAB, _SLAB), :],
                bufs[nxt], sems[nxt])
        copies[cur].wait()
        iref = bufs[cur]

        for gg in range(_SLAB):
            ggv = jnp.full((16,), gg, jnp.int32)

            def body(i, carry, iref=iref, ggv=ggv):
                a0, a1, tv = carry
                for j in range(_U):
                    iv = plsc.load_gather(iref, [ggv, tv])
                    w = plsc.load_gather(tbl_v, [iv])
                    v0 = plsc.bitcast(lax.shift_left(w, 16), jnp.float32)
                    v1 = plsc.bitcast(
                        lax.bitwise_and(w, jnp.int32(-65536)), jnp.float32)
                    a0 = a0 + v0
                    a1 = a1 + v1
                    tv = tv + 1
                return (a0, a1, tv)

            acc0, acc1, _ = lax.fori_loop(
                0, _S // _U, body, (zero, zero, offs))
            g = sb * _SLAB + gg
            out0_v[pl.ds(g * 16, 16)] = acc0
            out1_v[pl.ds(g * 16, 16)] = acc1

    pltpu.sync_copy(out0_v, out_hbm.at[pl.ds(base, _BPW)])
    pltpu.sync_copy(out1_v, out_hbm.at[pl.ds(_B + base, _BPW)])


_sc_bag = functools.partial(
    pl.kernel,
    out_type=jax.ShapeDtypeStruct((2 * _B,), jnp.float32),
    mesh=plsc.VectorSubcoreMesh(core_axis_name="c", subcore_axis_name="s"),
    compiler_params=pltpu.CompilerParams(needs_layout_passes=False),
    scratch_types=[
        pltpu.VMEM((_TLEN,), jnp.int32),
        pltpu.VMEM((_SLAB, 16 * _S), jnp.int32),
        pltpu.VMEM((_SLAB, 16 * _S), jnp.int32),
        pltpu.VMEM((_BPW,), jnp.float32),
        pltpu.VMEM((_BPW,), jnp.float32),
        pltpu.SemaphoreType.DMA,
        pltpu.SemaphoreType.DMA,
        pltpu.SemaphoreType.DMA,
    ],
)(_sc_bag_body)


def kernel(texts, embed, W1, b1, Wc, bc):
    tbl = _table_kernel(embed.T, W1, b1.reshape(-1, 1), Wc,
                        bc.reshape(1, -1))
    out = _sc_bag(tbl, texts.reshape(_B // 16, 16 * _S))
    return out.reshape(2, _B).T
